# baseline (speedup 1.0000x reference)
import jax
import jax.numpy as jnp
from jax.experimental import pallas as pl
from jax.experimental.pallas import tpu as pltpu

K = 50000
Q = 256
D = 6144
BLK = 400
_EPS = 1e-12
_RTOL = 1e-5
_ATOL = 1e-8


def _count_kernel(data_ref, q_ref, t_ref, out_ref, th_s, tol_s):
    @pl.when(pl.program_id(0) == 0)
    def _prep():
        q = q_ref[...]
        t = t_ref[...]
        qn = jnp.clip(jnp.sqrt(jnp.sum(q * q, axis=1, keepdims=True)),
                      _EPS, None)
        tn = jnp.clip(jnp.sqrt(jnp.sum(t * t, axis=1, keepdims=True)),
                      _EPS, None)
        th = jnp.sum(q * t, axis=1, keepdims=True) / (qn * tn)
        th_scaled = th * qn
        th_s[...] = th_scaled
        tol_s[...] = qn * _ATOL + _RTOL * jnp.abs(th_scaled)

    pT = jax.lax.dot_general(
        q_ref[...], data_ref[...],
        dimension_numbers=(((1,), (1,)), ((), ())),
        preferred_element_type=jnp.float32)
    th = th_s[...]
    mask = jnp.logical_or(pT >= th, jnp.abs(pT - th) <= tol_s[...])
    partial = jnp.sum(mask.astype(jnp.int32), axis=1, keepdims=True)

    @pl.when(pl.program_id(0) == 0)
    def _():
        out_ref[...] = partial - 1

    @pl.when(pl.program_id(0) != 0)
    def _():
        out_ref[...] += partial


def kernel(queries, truths, data, query_idx_in_rankingset,
           use_actaul_mw_for_retrival, use_jaccard):
    ct = pl.pallas_call(
        _count_kernel,
        grid=(K // BLK,),
        in_specs=[
            pl.BlockSpec((BLK, D), lambda i: (i, 0)),
            pl.BlockSpec((Q, D), lambda i: (0, 0)),
            pl.BlockSpec((Q, D), lambda i: (0, 0)),
        ],
        out_specs=pl.BlockSpec((Q, 1), lambda i: (0, 0)),
        out_shape=jax.ShapeDtypeStruct((Q, 1), jnp.int32),
        scratch_shapes=[
            pltpu.VMEM((Q, 1), jnp.float32),
            pltpu.VMEM((Q, 1), jnp.float32),
        ],
    )(data, queries, truths)
    return ct.reshape(1, Q)


# speedup vs baseline: 1.0327x; 1.0327x over previous
import jax
import jax.numpy as jnp
from jax.experimental import pallas as pl
from jax.experimental.pallas import tpu as pltpu

K = 50000
Q = 256
D = 6144
BLK = 400
_EPS = 1e-12
_RTOL = 1e-5
_ATOL = 1e-8


def _count_kernel(data_ref, q_ref, t_ref, out_ref, qT_s, th_s, tol_s):
    @pl.when(pl.program_id(0) == 0)
    def _prep():
        q = q_ref[...]
        t = t_ref[...]
        qn = jnp.clip(jnp.sqrt(jnp.sum(q * q, axis=1, keepdims=True)),
                      _EPS, None)
        tn = jnp.clip(jnp.sqrt(jnp.sum(t * t, axis=1, keepdims=True)),
                      _EPS, None)
        th = jnp.sum(q * t, axis=1, keepdims=True) / (qn * tn)
        th_scaled = th * qn
        tol = qn * _ATOL + _RTOL * jnp.abs(th_scaled)
        th_s[...] = th_scaled.T
        tol_s[...] = tol.T
        qT_s[...] = q.T

    p = jnp.dot(data_ref[...], qT_s[...], preferred_element_type=jnp.float32)
    th = th_s[...]
    mask = jnp.logical_or(p >= th, jnp.abs(p - th) <= tol_s[...])
    partial = jnp.sum(mask.astype(jnp.int32), axis=0, keepdims=True)

    @pl.when(pl.program_id(0) == 0)
    def _():
        out_ref[...] = partial - 1

    @pl.when(pl.program_id(0) != 0)
    def _():
        out_ref[...] += partial


def kernel(queries, truths, data, query_idx_in_rankingset,
           use_actaul_mw_for_retrival, use_jaccard):
    return pl.pallas_call(
        _count_kernel,
        grid=(K // BLK,),
        in_specs=[
            pl.BlockSpec((BLK, D), lambda i: (i, 0)),
            pl.BlockSpec((Q, D), lambda i: (0, 0)),
            pl.BlockSpec((Q, D), lambda i: (0, 0)),
        ],
        out_specs=pl.BlockSpec((1, Q), lambda i: (0, 0)),
        out_shape=jax.ShapeDtypeStruct((1, Q), jnp.int32),
        scratch_shapes=[
            pltpu.VMEM((D, Q), jnp.float32),
            pltpu.VMEM((1, Q), jnp.float32),
            pltpu.VMEM((1, Q), jnp.float32),
        ],
    )(data, queries, truths)
